# Initial kernel scaffold; baseline (speedup 1.0000x reference)
import jax
import jax.numpy as jnp
from jax.experimental import pallas as pl


def kernel(x, gate_W, gate_b, Wx, Vx, Wo):
    raise NotImplementedError("write your pallas kernel here")


# speedup vs baseline: 1.5364x; 1.5364x over previous
import functools

import jax
import jax.numpy as jnp
from jax import lax
from jax.experimental import pallas as pl
from jax.experimental.pallas import tpu as pltpu

T = 2048
D = 768
DFF = 3072
E = 8
FCH = 768
F = DFF // FCH


def _moe_body(x_ref, gw_ref, gb_ref, wx_ref, vx_ref, wo_ref,
              out_ref, logits_ref, coef_ref):
    e = pl.program_id(0)
    f = pl.program_id(1)

    @pl.when(jnp.logical_and(e == 0, f == 0))
    def _router():
        xv = x_ref[...]
        logits = lax.dot_general(
            xv, gw_ref[...], (((1,), (1,)), ((), ())),
            preferred_element_type=jnp.float32) + gb_ref[...]
        logits_ref[...] = logits
        lane = lax.broadcasted_iota(jnp.int32, (T, E), 1)
        m1 = jnp.max(logits, axis=-1, keepdims=True)
        a1 = jnp.min(jnp.where(logits == m1, lane, E), axis=-1, keepdims=True)
        masked = jnp.where(lane == a1, -jnp.inf, logits)
        m2 = jnp.max(masked, axis=-1, keepdims=True)
        a2 = jnp.min(jnp.where(masked == m2, lane, E), axis=-1, keepdims=True)
        w1 = 1.0 / (1.0 + jnp.exp(m2 - m1))
        w2 = 1.0 - w1
        coef_ref[...] = jnp.where(lane == a1, w1, 0.0) + jnp.where(lane == a2, w2, 0.0)

    @pl.when(jnp.logical_and(e == 0, f == 0))
    def _init():
        out_ref[...] = jnp.zeros_like(out_ref)

    xv = x_ref[...]
    a = lax.dot_general(xv, wx_ref[0], (((1,), (1,)), ((), ())),
                        preferred_element_type=jnp.float32)
    b = lax.dot_general(xv, vx_ref[0], (((1,), (1,)), ((), ())),
                        preferred_element_type=jnp.float32)
    h = a * jax.nn.sigmoid(a) * b
    o = lax.dot_general(h, wo_ref[0], (((1,), (1,)), ((), ())),
                        preferred_element_type=jnp.float32)
    lane = lax.broadcasted_iota(jnp.int32, (T, E), 1)
    col = jnp.sum(jnp.where(lane == e, coef_ref[...], 0.0), axis=-1,
                  keepdims=True)
    out_ref[...] += o * col


@functools.partial(jax.jit, static_argnames=())
def kernel(x, gate_W, gate_b, Wx, Vx, Wo):
    bsz, seq, d = x.shape
    tokens = x.reshape(-1, d)
    gb = gate_b.reshape(1, E)
    out, logits = pl.pallas_call(
        _moe_body,
        grid=(E, F),
        in_specs=[
            pl.BlockSpec((T, D), lambda e, f: (0, 0)),
            pl.BlockSpec((E, D), lambda e, f: (0, 0)),
            pl.BlockSpec((1, E), lambda e, f: (0, 0)),
            pl.BlockSpec((1, FCH, D), lambda e, f: (e, f, 0)),
            pl.BlockSpec((1, FCH, D), lambda e, f: (e, f, 0)),
            pl.BlockSpec((1, D, FCH), lambda e, f: (e, 0, f)),
        ],
        out_specs=[
            pl.BlockSpec((T, D), lambda e, f: (0, 0)),
            pl.BlockSpec((T, E), lambda e, f: (0, 0)),
        ],
        out_shape=[
            jax.ShapeDtypeStruct((T, D), jnp.float32),
            jax.ShapeDtypeStruct((T, E), jnp.float32),
        ],
        scratch_shapes=[pltpu.VMEM((T, E), jnp.float32)],
        compiler_params=pltpu.CompilerParams(
            dimension_semantics=("arbitrary", "arbitrary")),
    )(tokens, gate_W, gb, Wx, Vx, Wo)
    return out.reshape(bsz, seq, d), logits
